# Initial kernel scaffold; baseline (speedup 1.0000x reference)
import functools

import jax
import jax.numpy as jnp
from jax import lax
from jax.experimental import pallas as pl
from jax.experimental.pallas import tpu as pltpu
from jax.experimental.pallas import tpu_sc as plsc

N = 50000
EPS = 1e-5
C = 32
LANES = 16
EB = 128
SUPER = 1024
NTILES = 16
ROWS_PT = N // NTILES
PT = 50176
E_PAD = NTILES * PT
NSB = PT // SUPER


def _make_sc_agg(nc):
    mesh = plsc.VectorSubcoreMesh(core_axis_name="c", subcore_axis_name="s")

    @functools.partial(
        pl.kernel,
        mesh=mesh,
        out_type=jax.ShapeDtypeStruct((nc, N, C), jnp.float32),
        scratch_types=[
            pltpu.VMEM_SHARED((N, C), jnp.float32),
            pltpu.VMEM((SUPER,), jnp.int32),
            pltpu.VMEM((SUPER,), jnp.int32),
            pltpu.VMEM((SUPER,), jnp.float32),
            pltpu.VMEM((EB,), jnp.int32),
            pltpu.VMEM((EB,), jnp.int32),
            pltpu.VMEM((EB, C), jnp.float32),
            pltpu.SemaphoreType.DMA,
        ],
    )
    def sc_agg(hflat, src, dst, w, out, acc, srcb, dstb, wb, gidx, sidx, rows, sem):
        cid = lax.axis_index("c")
        sid = lax.axis_index("s")
        ebase = sid * PT
        r0 = sid * ROWS_PT
        nch = (nc - cid + 1) // 2

        def chunk_body(i, carry):
            ch = 2 * i + cid
            zv = jnp.zeros((LANES,), jnp.float32)

            def zrow(e, c2):
                rows[e, pl.ds(0, LANES)] = zv
                rows[e, pl.ds(LANES, LANES)] = zv
                return c2

            lax.fori_loop(0, EB, zrow, 0)
            nfull = ROWS_PT // EB
            rem = ROWS_PT - nfull * EB

            def zcp(z, c2):
                pltpu.sync_copy(rows, acc.at[pl.ds(r0 + z * EB, EB)])
                return c2

            lax.fori_loop(0, nfull, zcp, 0)
            pltpu.sync_copy(rows.at[pl.ds(0, rem)],
                            acc.at[pl.ds(r0 + nfull * EB, rem)])
            plsc.subcore_barrier()

            chn = jnp.full((LANES,), ch * N, jnp.int32)

            def super_body(sb, c2):
                off = ebase + sb * SUPER
                pltpu.sync_copy(src.at[pl.ds(off, SUPER)], srcb)
                pltpu.sync_copy(dst.at[pl.ds(off, SUPER)], dstb)
                pltpu.sync_copy(w.at[pl.ds(off, SUPER)], wb)
                for b in range(SUPER // EB):
                    for v in range(EB // LANES):
                        gidx[pl.ds(v * LANES, LANES)] = (
                            srcb[pl.ds(b * EB + v * LANES, LANES)] + chn)
                        sidx[pl.ds(v * LANES, LANES)] = (
                            dstb[pl.ds(b * EB + v * LANES, LANES)])
                    pltpu.async_copy(hflat.at[gidx], rows, sem).wait()

                    def scale(e, c3):
                        wv = wb[b * EB + e]
                        rows[e, pl.ds(0, LANES)] = rows[e, pl.ds(0, LANES)] * wv
                        rows[e, pl.ds(LANES, LANES)] = (
                            rows[e, pl.ds(LANES, LANES)] * wv)
                        return c3

                    lax.fori_loop(0, EB, scale, 0)
                    pltpu.sync_copy(rows, acc.at[sidx], add=True)
                return c2

            lax.fori_loop(0, NSB, super_body, 0)
            plsc.subcore_barrier()
            pltpu.sync_copy(acc.at[pl.ds(r0, ROWS_PT)],
                            out.at[ch, pl.ds(r0, ROWS_PT)])
            plsc.subcore_barrier()
            return carry

        lax.fori_loop(0, nch, chunk_body, 0)

    return sc_agg


_R = 2500


def _dot(a, b):
    return lax.dot_general(a, b, (((1,), (1,)), ((), ())),
                           preferred_element_type=jnp.float32)


def _l1_body(x_ref, W_ref, b_ref, Ws_ref, bs_ref, hc_ref, x2_ref):
    xb = x_ref[...]
    h = _dot(xb, W_ref[...]) + b_ref[...]
    x2 = _dot(xb, Ws_ref[...]) + bs_ref[...]
    nc = hc_ref.shape[0]
    hp = jnp.pad(h, ((0, 0), (0, nc * C - h.shape[1])))
    x2_ref[...] = jnp.pad(x2, ((0, 0), (0, x2_ref.shape[1] - x2.shape[1])))
    for c in range(nc):
        hc_ref[c] = hp[:, c * C:(c + 1) * C]


def _lB_body(aggc_ref, x2p_ref, W_ref, b_ref, Ws_ref, bs_ref, hc_ref, x2_ref):
    ncin = aggc_ref.shape[0]
    hin = jnp.concatenate(
        [jax.nn.relu(aggc_ref[c] + x2p_ref[:, c * C:(c + 1) * C])
         for c in range(ncin)], axis=1)
    h = _dot(hin, W_ref[...]) + b_ref[...]
    x2 = _dot(hin, Ws_ref[...]) + bs_ref[...]
    nc = hc_ref.shape[0]
    hp = jnp.pad(h, ((0, 0), (0, nc * C - h.shape[1])))
    x2_ref[...] = jnp.pad(x2, ((0, 0), (0, x2_ref.shape[1] - x2.shape[1])))
    for c in range(nc):
        hc_ref[c] = hp[:, c * C:(c + 1) * C]


def _head1_body(aggc_ref, x2p_ref, W1_ref, b1_ref, W2_ref, b2_ref,
                u_ref, st_ref):
    i = pl.program_id(0)
    ncin = aggc_ref.shape[0]
    h3 = jnp.concatenate(
        [jax.nn.relu(aggc_ref[c] + x2p_ref[:, c * C:(c + 1) * C])
         for c in range(ncin)], axis=1)
    t = jax.nn.relu(_dot(h3, W1_ref[...]) + b1_ref[...])
    u = _dot(t, W2_ref[...]) + b2_ref[...]
    u_ref[...] = u
    s0 = jnp.sum(u, axis=0, keepdims=True)
    s1 = jnp.sum(u * u, axis=0, keepdims=True)
    upd = jnp.concatenate([s0, s1, jnp.zeros((6, u.shape[1]), jnp.float32)],
                          axis=0)

    @pl.when(i == 0)
    def _():
        st_ref[...] = upd

    @pl.when(i > 0)
    def _():
        st_ref[...] = st_ref[...] + upd


def _head2_body(u_ref, sc_ref, sh_ref, W_ref, b_ref, o_ref):
    y = jax.nn.relu(u_ref[...] * sc_ref[...] + sh_ref[...])
    o_ref[...] = _dot(y, W_ref[...]) + b_ref[...]


def _full(shape):
    return pl.BlockSpec(shape, lambda i: (0,) * len(shape))


def _tc_layer1(x, W, b, Ws, bs, nc):
    Fop = nc * C
    grid = (N // _R,)
    return pl.pallas_call(
        _l1_body,
        grid=grid,
        in_specs=[
            pl.BlockSpec((_R, x.shape[1]), lambda i: (i, 0)),
            _full(W.shape), _full(b.shape), _full(Ws.shape), _full(bs.shape),
        ],
        out_specs=[
            pl.BlockSpec((nc, _R, C), lambda i: (0, i, 0)),
            pl.BlockSpec((_R, Fop), lambda i: (i, 0)),
        ],
        out_shape=[
            jax.ShapeDtypeStruct((nc, N, C), jnp.float32),
            jax.ShapeDtypeStruct((N, Fop), jnp.float32),
        ],
    )(x, W, b, Ws, bs)


def _tc_layerB(aggc, x2p, W, b, Ws, bs, nc):
    ncin = aggc.shape[0]
    Fop = nc * C
    grid = (N // _R,)
    return pl.pallas_call(
        _lB_body,
        grid=grid,
        in_specs=[
            pl.BlockSpec((ncin, _R, C), lambda i: (0, i, 0)),
            pl.BlockSpec((_R, x2p.shape[1]), lambda i: (i, 0)),
            _full(W.shape), _full(b.shape), _full(Ws.shape), _full(bs.shape),
        ],
        out_specs=[
            pl.BlockSpec((nc, _R, C), lambda i: (0, i, 0)),
            pl.BlockSpec((_R, Fop), lambda i: (i, 0)),
        ],
        out_shape=[
            jax.ShapeDtypeStruct((nc, N, C), jnp.float32),
            jax.ShapeDtypeStruct((N, Fop), jnp.float32),
        ],
    )(aggc, x2p, W, b, Ws, bs)


def _tc_head1(aggc, x2p, W1, b1, W2, b2):
    ncin = aggc.shape[0]
    Fo = W2.shape[0]
    grid = (N // _R,)
    return pl.pallas_call(
        _head1_body,
        grid=grid,
        in_specs=[
            pl.BlockSpec((ncin, _R, C), lambda i: (0, i, 0)),
            pl.BlockSpec((_R, x2p.shape[1]), lambda i: (i, 0)),
            _full(W1.shape), _full(b1.shape), _full(W2.shape), _full(b2.shape),
        ],
        out_specs=[
            pl.BlockSpec((_R, Fo), lambda i: (i, 0)),
            pl.BlockSpec((8, Fo), lambda i: (0, 0)),
        ],
        out_shape=[
            jax.ShapeDtypeStruct((N, Fo), jnp.float32),
            jax.ShapeDtypeStruct((8, Fo), jnp.float32),
        ],
    )(aggc, x2p, W1, b1, W2, b2)


def _tc_head2(u, scale, shift, W, b):
    Fo = W.shape[0]
    grid = (N // _R,)
    return pl.pallas_call(
        _head2_body,
        grid=grid,
        in_specs=[
            pl.BlockSpec((_R, u.shape[1]), lambda i: (i, 0)),
            _full(scale.shape), _full(shift.shape),
            _full(W.shape), _full(b.shape),
        ],
        out_specs=pl.BlockSpec((_R, Fo), lambda i: (i, 0)),
        out_shape=jax.ShapeDtypeStruct((N, Fo), jnp.float32),
    )(u, scale, shift, W, b)


_sc_agg7 = _make_sc_agg(7)
_sc_agg4 = _make_sc_agg(4)


def kernel(x, edge_weight, W1, b1, W1s, b1s, W2, b2, W2s, b2s, W3, b3,
           W3s, b3s, Wfc1, bfc1, Wfc2a, bfc2a, gamma, beta, Wfc2b, bfc2b,
           edge_index):
    f32 = jnp.float32
    src = edge_index[1].astype(jnp.int32)
    dstn = edge_index[0].astype(jnp.int32)
    npad = E_PAD - src.shape[0]
    srcp = jnp.concatenate([src, jnp.zeros((npad,), jnp.int32)])
    dstp = jnp.concatenate([dstn, jnp.zeros((npad,), jnp.int32)])
    wp = jnp.concatenate([edge_weight.astype(f32), jnp.zeros((npad,), f32)])

    r2 = lambda v: v.reshape(1, -1)

    hc1, x21 = _tc_layer1(x, W1, r2(b1), W1s, r2(b1s), nc=7)
    agg1 = _sc_agg7(hc1.reshape(7 * N, C), srcp, dstp, wp)

    W2p = jnp.pad(W2, ((0, 0), (0, 24)))
    W2sp = jnp.pad(W2s, ((0, 0), (0, 24)))
    hc2, x22 = _tc_layerB(agg1, x21, W2p, r2(b2), W2sp, r2(b2s), nc=4)
    agg2 = _sc_agg4(hc2.reshape(4 * N, C), srcp, dstp, wp)

    hc3, x23 = _tc_layerB(agg2, x22, W3, r2(b3), W3s, r2(b3s), nc=4)
    agg3 = _sc_agg4(hc3.reshape(4 * N, C), srcp, dstp, wp)

    u, st = _tc_head1(agg3, x23, Wfc1, r2(bfc1), Wfc2a, r2(bfc2a))
    mean = st[0] / N
    var = st[1] / N - mean * mean
    scale = gamma / jnp.sqrt(var + EPS)
    shift = beta - mean * scale
    return _tc_head2(u, r2(scale), r2(shift), Wfc2b, r2(bfc2b))


# speedup vs baseline: 2.2958x; 2.2958x over previous
import functools

import jax
import jax.numpy as jnp
from jax import lax
from jax.experimental import pallas as pl
from jax.experimental.pallas import tpu as pltpu
from jax.experimental.pallas import tpu_sc as plsc

N = 50000
EPS = 1e-5
C = 32
LANES = 16
EB = 128
SUPER = 1024
NTILES = 16
ROWS_PT = 3128
NPAD = NTILES * ROWS_PT
PT = 50176
E_PAD = NTILES * PT
NSB = PT // SUPER


@functools.lru_cache(maxsize=None)
def _make_sc_agg(nc):
    mesh = plsc.VectorSubcoreMesh(core_axis_name="c", subcore_axis_name="s")

    @functools.partial(
        pl.kernel,
        mesh=mesh,
        compiler_params=pltpu.CompilerParams(use_tc_tiling_on_sc=False),
        out_type=jax.ShapeDtypeStruct((nc, NPAD, C), jnp.float32),
        scratch_types=[
            pltpu.VMEM_SHARED((NPAD, C), jnp.float32),
            pltpu.VMEM((SUPER,), jnp.int32),
            pltpu.VMEM((SUPER,), jnp.int32),
            pltpu.VMEM((SUPER,), jnp.float32),
            pltpu.VMEM((EB,), jnp.int32),
            pltpu.VMEM((EB,), jnp.int32),
            pltpu.VMEM((EB, C), jnp.float32),
            pltpu.SemaphoreType.DMA,
        ],
    )
    def sc_agg(hflat, src, dst, w, out, acc, srcb, dstb, wb, gidx, sidx, rows, sem):
        cid = lax.axis_index("c")
        sid = lax.axis_index("s")
        ebase = sid * PT
        r0 = sid * ROWS_PT
        nch = (nc - cid + 1) // 2

        def chunk_body(i, carry):
            ch = 2 * i + cid
            zv = jnp.zeros((LANES,), jnp.float32)

            def zrow(e, c2):
                rows[e, pl.ds(0, LANES)] = zv
                rows[e, pl.ds(LANES, LANES)] = zv
                return c2

            lax.fori_loop(0, EB, zrow, 0)
            nfull = ROWS_PT // EB
            rem = ROWS_PT - nfull * EB

            def zcp(z, c2):
                pltpu.sync_copy(rows, acc.at[pl.ds(r0 + z * EB, EB)])
                return c2

            lax.fori_loop(0, nfull, zcp, 0)
            pltpu.sync_copy(rows.at[pl.ds(0, rem)],
                            acc.at[pl.ds(r0 + nfull * EB, rem)])
            plsc.subcore_barrier()

            chn = jnp.full((LANES,), ch * N, jnp.int32)

            def super_body(sb, c2):
                off = ebase + sb * SUPER
                pltpu.sync_copy(src.at[pl.ds(off, SUPER)], srcb)
                pltpu.sync_copy(dst.at[pl.ds(off, SUPER)], dstb)
                pltpu.sync_copy(w.at[pl.ds(off, SUPER)], wb)
                for b in range(SUPER // EB):
                    for v in range(EB // LANES):
                        gidx[pl.ds(v * LANES, LANES)] = (
                            srcb[pl.ds(b * EB + v * LANES, LANES)] + chn)
                        sidx[pl.ds(v * LANES, LANES)] = (
                            dstb[pl.ds(b * EB + v * LANES, LANES)])
                    pltpu.async_copy(hflat.at[gidx], rows, sem).wait()
                    for g in range(EB // LANES):
                        wv = wb[pl.ds(b * EB + g * LANES, LANES)]
                        for j in range(LANES):
                            e = g * LANES + j
                            s = wv[j]
                            rows[e, pl.ds(0, LANES)] = (
                                rows[e, pl.ds(0, LANES)] * s)
                            rows[e, pl.ds(LANES, LANES)] = (
                                rows[e, pl.ds(LANES, LANES)] * s)
                    pltpu.sync_copy(rows, acc.at[sidx], add=True)
                return c2

            lax.fori_loop(0, NSB, super_body, 0)
            plsc.subcore_barrier()
            pltpu.sync_copy(acc.at[pl.ds(r0, ROWS_PT)],
                            out.at[ch, pl.ds(r0, ROWS_PT)])
            plsc.subcore_barrier()
            return carry

        lax.fori_loop(0, nch, chunk_body, 0)

    return sc_agg


_R = 2000


def _dot(a, b):
    return lax.dot_general(a, b, (((1,), (1,)), ((), ())),
                           preferred_element_type=jnp.float32)


def _l1_body(x_ref, W_ref, b_ref, Ws_ref, bs_ref, hc_ref, x2_ref):
    xb = x_ref[...]
    h = _dot(xb, W_ref[...]) + b_ref[...]
    x2 = _dot(xb, Ws_ref[...]) + bs_ref[...]
    nc = hc_ref.shape[0]
    hp = jnp.pad(h, ((0, 0), (0, nc * C - h.shape[1])))
    x2_ref[...] = jnp.pad(x2, ((0, 0), (0, x2_ref.shape[1] - x2.shape[1])))
    for c in range(nc):
        hc_ref[c] = hp[:, c * C:(c + 1) * C]


def _lB_body(aggc_ref, x2p_ref, W_ref, b_ref, Ws_ref, bs_ref, hc_ref, x2_ref):
    ncin = aggc_ref.shape[0]
    hin = jnp.concatenate(
        [jax.nn.relu(aggc_ref[c] + x2p_ref[:, c * C:(c + 1) * C])
         for c in range(ncin)], axis=1)
    h = _dot(hin, W_ref[...]) + b_ref[...]
    x2 = _dot(hin, Ws_ref[...]) + bs_ref[...]
    nc = hc_ref.shape[0]
    hp = jnp.pad(h, ((0, 0), (0, nc * C - h.shape[1])))
    x2_ref[...] = jnp.pad(x2, ((0, 0), (0, x2_ref.shape[1] - x2.shape[1])))
    for c in range(nc):
        hc_ref[c] = hp[:, c * C:(c + 1) * C]


def _head1_body(aggc_ref, x2p_ref, W1_ref, b1_ref, W2_ref, b2_ref,
                u_ref, st_ref):
    i = pl.program_id(0)
    ncin = aggc_ref.shape[0]
    h3 = jnp.concatenate(
        [jax.nn.relu(aggc_ref[c] + x2p_ref[:, c * C:(c + 1) * C])
         for c in range(ncin)], axis=1)
    t = jax.nn.relu(_dot(h3, W1_ref[...]) + b1_ref[...])
    u = _dot(t, W2_ref[...]) + b2_ref[...]
    u_ref[...] = u
    s0 = jnp.sum(u, axis=0, keepdims=True)
    s1 = jnp.sum(u * u, axis=0, keepdims=True)
    upd = jnp.concatenate([s0, s1, jnp.zeros((6, u.shape[1]), jnp.float32)],
                          axis=0)

    @pl.when(i == 0)
    def _():
        st_ref[...] = upd

    @pl.when(i > 0)
    def _():
        st_ref[...] = st_ref[...] + upd


def _head2_body(u_ref, sc_ref, sh_ref, W_ref, b_ref, o_ref):
    y = jax.nn.relu(u_ref[...] * sc_ref[...] + sh_ref[...])
    o_ref[...] = _dot(y, W_ref[...]) + b_ref[...]


def _full(shape):
    return pl.BlockSpec(shape, lambda i: (0,) * len(shape))


def _tc_layer1(x, W, b, Ws, bs, nc):
    Fop = nc * C
    grid = (N // _R,)
    return pl.pallas_call(
        _l1_body,
        grid=grid,
        in_specs=[
            pl.BlockSpec((_R, x.shape[1]), lambda i: (i, 0)),
            _full(W.shape), _full(b.shape), _full(Ws.shape), _full(bs.shape),
        ],
        out_specs=[
            pl.BlockSpec((nc, _R, C), lambda i: (0, i, 0)),
            pl.BlockSpec((_R, Fop), lambda i: (i, 0)),
        ],
        out_shape=[
            jax.ShapeDtypeStruct((nc, N, C), jnp.float32),
            jax.ShapeDtypeStruct((N, Fop), jnp.float32),
        ],
    )(x, W, b, Ws, bs)


def _tc_layerB(aggc, x2p, W, b, Ws, bs, nc):
    ncin = aggc.shape[0]
    Fop = nc * C
    grid = (N // _R,)
    return pl.pallas_call(
        _lB_body,
        grid=grid,
        in_specs=[
            pl.BlockSpec((ncin, _R, C), lambda i: (0, i, 0)),
            pl.BlockSpec((_R, x2p.shape[1]), lambda i: (i, 0)),
            _full(W.shape), _full(b.shape), _full(Ws.shape), _full(bs.shape),
        ],
        out_specs=[
            pl.BlockSpec((nc, _R, C), lambda i: (0, i, 0)),
            pl.BlockSpec((_R, Fop), lambda i: (i, 0)),
        ],
        out_shape=[
            jax.ShapeDtypeStruct((nc, N, C), jnp.float32),
            jax.ShapeDtypeStruct((N, Fop), jnp.float32),
        ],
    )(aggc, x2p, W, b, Ws, bs)


def _tc_head1(aggc, x2p, W1, b1, W2, b2):
    ncin = aggc.shape[0]
    Fo = W2.shape[0]
    grid = (N // _R,)
    return pl.pallas_call(
        _head1_body,
        grid=grid,
        in_specs=[
            pl.BlockSpec((ncin, _R, C), lambda i: (0, i, 0)),
            pl.BlockSpec((_R, x2p.shape[1]), lambda i: (i, 0)),
            _full(W1.shape), _full(b1.shape), _full(W2.shape), _full(b2.shape),
        ],
        out_specs=[
            pl.BlockSpec((_R, Fo), lambda i: (i, 0)),
            pl.BlockSpec((8, Fo), lambda i: (0, 0)),
        ],
        out_shape=[
            jax.ShapeDtypeStruct((N, Fo), jnp.float32),
            jax.ShapeDtypeStruct((8, Fo), jnp.float32),
        ],
    )(aggc, x2p, W1, b1, W2, b2)


def _tc_head2(u, scale, shift, W, b):
    Fo = W.shape[0]
    grid = (N // _R,)
    return pl.pallas_call(
        _head2_body,
        grid=grid,
        in_specs=[
            pl.BlockSpec((_R, u.shape[1]), lambda i: (i, 0)),
            _full(scale.shape), _full(shift.shape),
            _full(W.shape), _full(b.shape),
        ],
        out_specs=pl.BlockSpec((_R, Fo), lambda i: (i, 0)),
        out_shape=jax.ShapeDtypeStruct((N, Fo), jnp.float32),
    )(u, scale, shift, W, b)


def _sc_agg(hflat, src, dst, w, nc):
    return _make_sc_agg(nc)(hflat, src, dst, w)


def kernel(x, edge_weight, W1, b1, W1s, b1s, W2, b2, W2s, b2s, W3, b3,
           W3s, b3s, Wfc1, bfc1, Wfc2a, bfc2a, gamma, beta, Wfc2b, bfc2b,
           edge_index):
    f32 = jnp.float32
    src = edge_index[1].astype(jnp.int32)
    dstn = edge_index[0].astype(jnp.int32)
    npad = E_PAD - src.shape[0]
    srcp = jnp.concatenate([src, jnp.zeros((npad,), jnp.int32)])
    dstp = jnp.concatenate([dstn, jnp.zeros((npad,), jnp.int32)])
    wp = jnp.concatenate([edge_weight.astype(f32), jnp.zeros((npad,), f32)])

    r2 = lambda v: v.reshape(1, -1)

    hc1, x21 = _tc_layer1(x, W1, r2(b1), W1s, r2(b1s), nc=7)
    agg1 = _sc_agg(hc1.reshape(7 * N, C), srcp, dstp, wp, 7)

    W2p = jnp.pad(W2, ((0, 0), (0, 24)))
    W2sp = jnp.pad(W2s, ((0, 0), (0, 24)))
    hc2, x22 = _tc_layerB(agg1, x21, W2p, r2(b2), W2sp, r2(b2s), nc=4)
    agg2 = _sc_agg(hc2.reshape(4 * N, C), srcp, dstp, wp, 4)

    hc3, x23 = _tc_layerB(agg2, x22, W3, r2(b3), W3s, r2(b3s), nc=4)
    agg3 = _sc_agg(hc3.reshape(4 * N, C), srcp, dstp, wp, 4)

    u, st = _tc_head1(agg3, x23, Wfc1, r2(bfc1), Wfc2a, r2(bfc2a))
    mean = st[0] / N
    var = st[1] / N - mean * mean
    scale = gamma / jnp.sqrt(var + EPS)
    shift = beta - mean * scale
    return _tc_head2(u, r2(scale), r2(shift), Wfc2b, r2(bfc2b))


# speedup vs baseline: 3.0973x; 1.3491x over previous
import functools

import jax
import jax.numpy as jnp
from jax import lax
from jax.experimental import pallas as pl
from jax.experimental.pallas import tpu as pltpu
from jax.experimental.pallas import tpu_sc as plsc

N = 50000
EPS = 1e-5
C = 32
LANES = 16
EB = 128
SUPER = 1024
NTILES = 16
ROWS_PT = 3128
NPAD = NTILES * ROWS_PT
PT = 50176
E_PAD = NTILES * PT
NSB = PT // SUPER


@functools.lru_cache(maxsize=None)
def _make_sc_agg(nc):
    mesh = plsc.VectorSubcoreMesh(core_axis_name="c", subcore_axis_name="s")

    @functools.partial(
        pl.kernel,
        mesh=mesh,
        compiler_params=pltpu.CompilerParams(use_tc_tiling_on_sc=False),
        out_type=jax.ShapeDtypeStruct((nc, NPAD, C), jnp.float32),
        scratch_types=[
            pltpu.VMEM_SHARED((NPAD, C), jnp.float32),
            pltpu.VMEM((SUPER,), jnp.int32),
            pltpu.VMEM((SUPER,), jnp.int32),
            pltpu.VMEM((SUPER,), jnp.float32),
            pltpu.VMEM((EB,), jnp.int32),
            pltpu.VMEM((EB,), jnp.int32),
            pltpu.VMEM((EB,), jnp.int32),
            pltpu.VMEM((EB,), jnp.int32),
            pltpu.VMEM((EB, C), jnp.float32),
            pltpu.VMEM((EB, C), jnp.float32),
            pltpu.SemaphoreType.DMA,
            pltpu.SemaphoreType.DMA,
        ],
    )
    def sc_agg(hflat, src, dst, w, out, acc, srcb, dstb, wb,
               gidx0, gidx1, sidx0, sidx1, rows0, rows1, sem0, sem1):
        gx = (gidx0, gidx1)
        sx = (sidx0, sidx1)
        rw = (rows0, rows1)
        sm = (sem0, sem1)
        cid = lax.axis_index("c")
        sid = lax.axis_index("s")
        ebase = sid * PT
        r0 = sid * ROWS_PT
        nch = (nc - cid + 1) // 2

        def chunk_body(i, carry):
            ch = 2 * i + cid
            zv = jnp.zeros((LANES,), jnp.float32)

            def zrow(e, c2):
                rows0[e, pl.ds(0, LANES)] = zv
                rows0[e, pl.ds(LANES, LANES)] = zv
                return c2

            lax.fori_loop(0, EB, zrow, 0)
            nfull = ROWS_PT // EB
            rem = ROWS_PT - nfull * EB

            def zcp(z, c2):
                pltpu.sync_copy(rows0, acc.at[pl.ds(r0 + z * EB, EB)])
                return c2

            lax.fori_loop(0, nfull, zcp, 0)
            pltpu.sync_copy(rows0.at[pl.ds(0, rem)],
                            acc.at[pl.ds(r0 + nfull * EB, rem)])
            plsc.subcore_barrier()

            chn = jnp.full((LANES,), ch * N, jnp.int32)

            nb = SUPER // EB

            def super_body(sb, c2):
                off = ebase + sb * SUPER
                pltpu.sync_copy(src.at[pl.ds(off, SUPER)], srcb)
                pltpu.sync_copy(dst.at[pl.ds(off, SUPER)], dstb)
                pltpu.sync_copy(w.at[pl.ds(off, SUPER)], wb)

                def build(b, p):
                    for v in range(EB // LANES):
                        gx[p][pl.ds(v * LANES, LANES)] = (
                            srcb[pl.ds(b * EB + v * LANES, LANES)] + chn)
                        sx[p][pl.ds(v * LANES, LANES)] = (
                            dstb[pl.ds(b * EB + v * LANES, LANES)])

                build(0, 0)
                pend = [None, None]
                pend[0] = pltpu.async_copy(hflat.at[gx[0]], rw[0], sm[0])
                for b in range(nb):
                    p = b & 1
                    if b + 1 < nb:
                        build(b + 1, 1 - p)
                        pend[1 - p] = pltpu.async_copy(
                            hflat.at[gx[1 - p]], rw[1 - p], sm[1 - p])
                    pend[p].wait()
                    rows = rw[p]
                    for g in range(EB // LANES):
                        wv = wb[pl.ds(b * EB + g * LANES, LANES)]
                        for j in range(LANES):
                            e = g * LANES + j
                            s = wv[j]
                            rows[e, pl.ds(0, LANES)] = (
                                rows[e, pl.ds(0, LANES)] * s)
                            rows[e, pl.ds(LANES, LANES)] = (
                                rows[e, pl.ds(LANES, LANES)] * s)
                    pltpu.sync_copy(rows, acc.at[sx[p]], add=True)
                return c2

            lax.fori_loop(0, NSB, super_body, 0)
            plsc.subcore_barrier()
            pltpu.sync_copy(acc.at[pl.ds(r0, ROWS_PT)],
                            out.at[ch, pl.ds(r0, ROWS_PT)])
            plsc.subcore_barrier()
            return carry

        lax.fori_loop(0, nch, chunk_body, 0)

    return sc_agg


_R = 2000


def _dot(a, b):
    return lax.dot_general(a, b, (((1,), (1,)), ((), ())),
                           preferred_element_type=jnp.float32)


def _l1_body(x_ref, W_ref, b_ref, Ws_ref, bs_ref, hc_ref, x2_ref):
    xb = x_ref[...]
    h = _dot(xb, W_ref[...]) + b_ref[...]
    x2 = _dot(xb, Ws_ref[...]) + bs_ref[...]
    nc = hc_ref.shape[0]
    hp = jnp.pad(h, ((0, 0), (0, nc * C - h.shape[1])))
    x2_ref[...] = jnp.pad(x2, ((0, 0), (0, x2_ref.shape[1] - x2.shape[1])))
    for c in range(nc):
        hc_ref[c] = hp[:, c * C:(c + 1) * C]


def _lB_body(aggc_ref, x2p_ref, W_ref, b_ref, Ws_ref, bs_ref, hc_ref, x2_ref):
    ncin = aggc_ref.shape[0]
    hin = jnp.concatenate(
        [jax.nn.relu(aggc_ref[c] + x2p_ref[:, c * C:(c + 1) * C])
         for c in range(ncin)], axis=1)
    h = _dot(hin, W_ref[...]) + b_ref[...]
    x2 = _dot(hin, Ws_ref[...]) + bs_ref[...]
    nc = hc_ref.shape[0]
    hp = jnp.pad(h, ((0, 0), (0, nc * C - h.shape[1])))
    x2_ref[...] = jnp.pad(x2, ((0, 0), (0, x2_ref.shape[1] - x2.shape[1])))
    for c in range(nc):
        hc_ref[c] = hp[:, c * C:(c + 1) * C]


def _head1_body(aggc_ref, x2p_ref, W1_ref, b1_ref, W2_ref, b2_ref,
                u_ref, st_ref):
    i = pl.program_id(0)
    ncin = aggc_ref.shape[0]
    h3 = jnp.concatenate(
        [jax.nn.relu(aggc_ref[c] + x2p_ref[:, c * C:(c + 1) * C])
         for c in range(ncin)], axis=1)
    t = jax.nn.relu(_dot(h3, W1_ref[...]) + b1_ref[...])
    u = _dot(t, W2_ref[...]) + b2_ref[...]
    u_ref[...] = u
    s0 = jnp.sum(u, axis=0, keepdims=True)
    s1 = jnp.sum(u * u, axis=0, keepdims=True)
    upd = jnp.concatenate([s0, s1, jnp.zeros((6, u.shape[1]), jnp.float32)],
                          axis=0)

    @pl.when(i == 0)
    def _():
        st_ref[...] = upd

    @pl.when(i > 0)
    def _():
        st_ref[...] = st_ref[...] + upd


def _head2_body(u_ref, sc_ref, sh_ref, W_ref, b_ref, o_ref):
    y = jax.nn.relu(u_ref[...] * sc_ref[...] + sh_ref[...])
    o_ref[...] = _dot(y, W_ref[...]) + b_ref[...]


def _full(shape):
    return pl.BlockSpec(shape, lambda i: (0,) * len(shape))


def _tc_layer1(x, W, b, Ws, bs, nc):
    Fop = nc * C
    grid = (N // _R,)
    return pl.pallas_call(
        _l1_body,
        grid=grid,
        in_specs=[
            pl.BlockSpec((_R, x.shape[1]), lambda i: (i, 0)),
            _full(W.shape), _full(b.shape), _full(Ws.shape), _full(bs.shape),
        ],
        out_specs=[
            pl.BlockSpec((nc, _R, C), lambda i: (0, i, 0)),
            pl.BlockSpec((_R, Fop), lambda i: (i, 0)),
        ],
        out_shape=[
            jax.ShapeDtypeStruct((nc, N, C), jnp.float32),
            jax.ShapeDtypeStruct((N, Fop), jnp.float32),
        ],
    )(x, W, b, Ws, bs)


def _tc_layerB(aggc, x2p, W, b, Ws, bs, nc):
    ncin = aggc.shape[0]
    Fop = nc * C
    grid = (N // _R,)
    return pl.pallas_call(
        _lB_body,
        grid=grid,
        in_specs=[
            pl.BlockSpec((ncin, _R, C), lambda i: (0, i, 0)),
            pl.BlockSpec((_R, x2p.shape[1]), lambda i: (i, 0)),
            _full(W.shape), _full(b.shape), _full(Ws.shape), _full(bs.shape),
        ],
        out_specs=[
            pl.BlockSpec((nc, _R, C), lambda i: (0, i, 0)),
            pl.BlockSpec((_R, Fop), lambda i: (i, 0)),
        ],
        out_shape=[
            jax.ShapeDtypeStruct((nc, N, C), jnp.float32),
            jax.ShapeDtypeStruct((N, Fop), jnp.float32),
        ],
    )(aggc, x2p, W, b, Ws, bs)


def _tc_head1(aggc, x2p, W1, b1, W2, b2):
    ncin = aggc.shape[0]
    Fo = W2.shape[0]
    grid = (N // _R,)
    return pl.pallas_call(
        _head1_body,
        grid=grid,
        in_specs=[
            pl.BlockSpec((ncin, _R, C), lambda i: (0, i, 0)),
            pl.BlockSpec((_R, x2p.shape[1]), lambda i: (i, 0)),
            _full(W1.shape), _full(b1.shape), _full(W2.shape), _full(b2.shape),
        ],
        out_specs=[
            pl.BlockSpec((_R, Fo), lambda i: (i, 0)),
            pl.BlockSpec((8, Fo), lambda i: (0, 0)),
        ],
        out_shape=[
            jax.ShapeDtypeStruct((N, Fo), jnp.float32),
            jax.ShapeDtypeStruct((8, Fo), jnp.float32),
        ],
    )(aggc, x2p, W1, b1, W2, b2)


def _tc_head2(u, scale, shift, W, b):
    Fo = W.shape[0]
    grid = (N // _R,)
    return pl.pallas_call(
        _head2_body,
        grid=grid,
        in_specs=[
            pl.BlockSpec((_R, u.shape[1]), lambda i: (i, 0)),
            _full(scale.shape), _full(shift.shape),
            _full(W.shape), _full(b.shape),
        ],
        out_specs=pl.BlockSpec((_R, Fo), lambda i: (i, 0)),
        out_shape=jax.ShapeDtypeStruct((N, Fo), jnp.float32),
    )(u, scale, shift, W, b)


def _sc_agg(hflat, src, dst, w, nc):
    return _make_sc_agg(nc)(hflat, src, dst, w)


def kernel(x, edge_weight, W1, b1, W1s, b1s, W2, b2, W2s, b2s, W3, b3,
           W3s, b3s, Wfc1, bfc1, Wfc2a, bfc2a, gamma, beta, Wfc2b, bfc2b,
           edge_index):
    f32 = jnp.float32
    src = edge_index[1].astype(jnp.int32)
    dstn = edge_index[0].astype(jnp.int32)
    npad = E_PAD - src.shape[0]
    srcp = jnp.concatenate([src, jnp.zeros((npad,), jnp.int32)])
    dstp = jnp.concatenate([dstn, jnp.zeros((npad,), jnp.int32)])
    wp = jnp.concatenate([edge_weight.astype(f32), jnp.zeros((npad,), f32)])

    r2 = lambda v: v.reshape(1, -1)

    hc1, x21 = _tc_layer1(x, W1, r2(b1), W1s, r2(b1s), nc=7)
    agg1 = _sc_agg(hc1.reshape(7 * N, C), srcp, dstp, wp, 7)

    W2p = jnp.pad(W2, ((0, 0), (0, 24)))
    W2sp = jnp.pad(W2s, ((0, 0), (0, 24)))
    hc2, x22 = _tc_layerB(agg1, x21, W2p, r2(b2), W2sp, r2(b2s), nc=4)
    agg2 = _sc_agg(hc2.reshape(4 * N, C), srcp, dstp, wp, 4)

    hc3, x23 = _tc_layerB(agg2, x22, W3, r2(b3), W3s, r2(b3s), nc=4)
    agg3 = _sc_agg(hc3.reshape(4 * N, C), srcp, dstp, wp, 4)

    u, st = _tc_head1(agg3, x23, Wfc1, r2(bfc1), Wfc2a, r2(bfc2a))
    mean = st[0] / N
    var = st[1] / N - mean * mean
    scale = gamma / jnp.sqrt(var + EPS)
    shift = beta - mean * scale
    return _tc_head2(u, r2(scale), r2(shift), Wfc2b, r2(bfc2b))


# speedup vs baseline: 3.2329x; 1.0438x over previous
import functools

import jax
import jax.numpy as jnp
from jax import lax
from jax.experimental import pallas as pl
from jax.experimental.pallas import tpu as pltpu
from jax.experimental.pallas import tpu_sc as plsc

N = 50000
EPS = 1e-5
C = 32
LANES = 16
EB = 128
SUPER = 1024
NTILES = 16
ROWS_PT = 3128
NPAD = NTILES * ROWS_PT
PT = 50176
E_PAD = NTILES * PT
NSB = PT // SUPER


@functools.lru_cache(maxsize=None)
def _make_sc_agg(nc):
    mesh = plsc.VectorSubcoreMesh(core_axis_name="c", subcore_axis_name="s")

    @functools.partial(
        pl.kernel,
        mesh=mesh,
        compiler_params=pltpu.CompilerParams(use_tc_tiling_on_sc=False),
        out_type=jax.ShapeDtypeStruct((nc, NPAD, C), jnp.float32),
        scratch_types=[
            pltpu.VMEM_SHARED((NPAD, C), jnp.float32),
            pltpu.VMEM((SUPER,), jnp.int32),
            pltpu.VMEM((SUPER,), jnp.int32),
            pltpu.VMEM((SUPER,), jnp.float32),
            pltpu.VMEM((EB,), jnp.int32),
            pltpu.VMEM((EB,), jnp.int32),
            pltpu.VMEM((EB,), jnp.int32),
            pltpu.VMEM((EB,), jnp.int32),
            pltpu.VMEM((EB,), jnp.int32),
            pltpu.VMEM((EB,), jnp.int32),
            pltpu.VMEM((EB, C), jnp.float32),
            pltpu.VMEM((EB, C), jnp.float32),
            pltpu.VMEM((EB, C), jnp.float32),
            pltpu.SemaphoreType.DMA,
            pltpu.SemaphoreType.DMA,
            pltpu.SemaphoreType.DMA,
            pltpu.SemaphoreType.DMA,
            pltpu.SemaphoreType.DMA,
            pltpu.SemaphoreType.DMA,
        ],
    )
    def sc_agg(hflat, src, dst, w, out, acc, srcb, dstb, wb,
               gidx0, gidx1, gidx2, sidx0, sidx1, sidx2,
               rows0, rows1, rows2, gs0, gs1, gs2, ss0, ss1, ss2):
        gx = (gidx0, gidx1, gidx2)
        sx = (sidx0, sidx1, sidx2)
        rw = (rows0, rows1, rows2)
        gsm = (gs0, gs1, gs2)
        ssm = (ss0, ss1, ss2)
        cid = lax.axis_index("c")
        sid = lax.axis_index("s")
        ebase = sid * PT
        r0 = sid * ROWS_PT
        nch = (nc - cid + 1) // 2

        def chunk_body(i, carry):
            ch = 2 * i + cid
            zv = jnp.zeros((LANES,), jnp.float32)

            def zrow(e, c2):
                rows0[e, pl.ds(0, LANES)] = zv
                rows0[e, pl.ds(LANES, LANES)] = zv
                return c2

            lax.fori_loop(0, EB, zrow, 0)
            nfull = ROWS_PT // EB
            rem = ROWS_PT - nfull * EB

            def zcp(z, c2):
                pltpu.sync_copy(rows0, acc.at[pl.ds(r0 + z * EB, EB)])
                return c2

            lax.fori_loop(0, nfull, zcp, 0)
            pltpu.sync_copy(rows0.at[pl.ds(0, rem)],
                            acc.at[pl.ds(r0 + nfull * EB, rem)])
            plsc.subcore_barrier()

            chn = jnp.full((LANES,), ch * N, jnp.int32)

            nb = SUPER // EB

            def super_body(sb, c2):
                off = ebase + sb * SUPER
                pltpu.sync_copy(src.at[pl.ds(off, SUPER)], srcb)
                pltpu.sync_copy(dst.at[pl.ds(off, SUPER)], dstb)
                pltpu.sync_copy(w.at[pl.ds(off, SUPER)], wb)

                def build(b, p):
                    for v in range(EB // LANES):
                        gx[p][pl.ds(v * LANES, LANES)] = (
                            srcb[pl.ds(b * EB + v * LANES, LANES)] + chn)
                        sx[p][pl.ds(v * LANES, LANES)] = (
                            dstb[pl.ds(b * EB + v * LANES, LANES)])

                build(0, 0)
                G = [None, None, None]
                S = [None, None, None]
                G[0] = pltpu.async_copy(hflat.at[gx[0]], rw[0], gsm[0])
                for b in range(nb):
                    p = b % 3
                    if b + 1 < nb:
                        q = (b + 1) % 3
                        if S[q] is not None:
                            S[q].wait()
                        build(b + 1, q)
                        G[q] = pltpu.async_copy(hflat.at[gx[q]], rw[q], gsm[q])
                    G[p].wait()
                    rows = rw[p]
                    for g in range(EB // LANES):
                        wv = wb[pl.ds(b * EB + g * LANES, LANES)]
                        for j in range(LANES):
                            e = g * LANES + j
                            s = wv[j]
                            rows[e, pl.ds(0, LANES)] = (
                                rows[e, pl.ds(0, LANES)] * s)
                            rows[e, pl.ds(LANES, LANES)] = (
                                rows[e, pl.ds(LANES, LANES)] * s)
                    S[p] = pltpu.async_copy(rows, acc.at[sx[p]], ssm[p],
                                            add=True)
                for b in range(nb - 3, nb):
                    S[b % 3].wait()
                return c2

            lax.fori_loop(0, NSB, super_body, 0)
            plsc.subcore_barrier()
            pltpu.sync_copy(acc.at[pl.ds(r0, ROWS_PT)],
                            out.at[ch, pl.ds(r0, ROWS_PT)])
            plsc.subcore_barrier()
            return carry

        lax.fori_loop(0, nch, chunk_body, 0)

    return sc_agg


_R = 2000


def _dot(a, b):
    return lax.dot_general(a, b, (((1,), (1,)), ((), ())),
                           preferred_element_type=jnp.float32)


def _l1_body(x_ref, W_ref, b_ref, Ws_ref, bs_ref, hc_ref, x2_ref):
    xb = x_ref[...]
    h = _dot(xb, W_ref[...]) + b_ref[...]
    x2 = _dot(xb, Ws_ref[...]) + bs_ref[...]
    nc = hc_ref.shape[0]
    hp = jnp.pad(h, ((0, 0), (0, nc * C - h.shape[1])))
    x2_ref[...] = jnp.pad(x2, ((0, 0), (0, x2_ref.shape[1] - x2.shape[1])))
    for c in range(nc):
        hc_ref[c] = hp[:, c * C:(c + 1) * C]


def _lB_body(aggc_ref, x2p_ref, W_ref, b_ref, Ws_ref, bs_ref, hc_ref, x2_ref):
    ncin = aggc_ref.shape[0]
    hin = jnp.concatenate(
        [jax.nn.relu(aggc_ref[c] + x2p_ref[:, c * C:(c + 1) * C])
         for c in range(ncin)], axis=1)
    h = _dot(hin, W_ref[...]) + b_ref[...]
    x2 = _dot(hin, Ws_ref[...]) + bs_ref[...]
    nc = hc_ref.shape[0]
    hp = jnp.pad(h, ((0, 0), (0, nc * C - h.shape[1])))
    x2_ref[...] = jnp.pad(x2, ((0, 0), (0, x2_ref.shape[1] - x2.shape[1])))
    for c in range(nc):
        hc_ref[c] = hp[:, c * C:(c + 1) * C]


def _head1_body(aggc_ref, x2p_ref, W1_ref, b1_ref, W2_ref, b2_ref,
                u_ref, st_ref):
    i = pl.program_id(0)
    ncin = aggc_ref.shape[0]
    h3 = jnp.concatenate(
        [jax.nn.relu(aggc_ref[c] + x2p_ref[:, c * C:(c + 1) * C])
         for c in range(ncin)], axis=1)
    t = jax.nn.relu(_dot(h3, W1_ref[...]) + b1_ref[...])
    u = _dot(t, W2_ref[...]) + b2_ref[...]
    u_ref[...] = u
    s0 = jnp.sum(u, axis=0, keepdims=True)
    s1 = jnp.sum(u * u, axis=0, keepdims=True)
    upd = jnp.concatenate([s0, s1, jnp.zeros((6, u.shape[1]), jnp.float32)],
                          axis=0)

    @pl.when(i == 0)
    def _():
        st_ref[...] = upd

    @pl.when(i > 0)
    def _():
        st_ref[...] = st_ref[...] + upd


def _head2_body(u_ref, sc_ref, sh_ref, W_ref, b_ref, o_ref):
    y = jax.nn.relu(u_ref[...] * sc_ref[...] + sh_ref[...])
    o_ref[...] = _dot(y, W_ref[...]) + b_ref[...]


def _full(shape):
    return pl.BlockSpec(shape, lambda i: (0,) * len(shape))


def _tc_layer1(x, W, b, Ws, bs, nc):
    Fop = nc * C
    grid = (N // _R,)
    return pl.pallas_call(
        _l1_body,
        grid=grid,
        in_specs=[
            pl.BlockSpec((_R, x.shape[1]), lambda i: (i, 0)),
            _full(W.shape), _full(b.shape), _full(Ws.shape), _full(bs.shape),
        ],
        out_specs=[
            pl.BlockSpec((nc, _R, C), lambda i: (0, i, 0)),
            pl.BlockSpec((_R, Fop), lambda i: (i, 0)),
        ],
        out_shape=[
            jax.ShapeDtypeStruct((nc, N, C), jnp.float32),
            jax.ShapeDtypeStruct((N, Fop), jnp.float32),
        ],
    )(x, W, b, Ws, bs)


def _tc_layerB(aggc, x2p, W, b, Ws, bs, nc):
    ncin = aggc.shape[0]
    Fop = nc * C
    grid = (N // _R,)
    return pl.pallas_call(
        _lB_body,
        grid=grid,
        in_specs=[
            pl.BlockSpec((ncin, _R, C), lambda i: (0, i, 0)),
            pl.BlockSpec((_R, x2p.shape[1]), lambda i: (i, 0)),
            _full(W.shape), _full(b.shape), _full(Ws.shape), _full(bs.shape),
        ],
        out_specs=[
            pl.BlockSpec((nc, _R, C), lambda i: (0, i, 0)),
            pl.BlockSpec((_R, Fop), lambda i: (i, 0)),
        ],
        out_shape=[
            jax.ShapeDtypeStruct((nc, N, C), jnp.float32),
            jax.ShapeDtypeStruct((N, Fop), jnp.float32),
        ],
    )(aggc, x2p, W, b, Ws, bs)


def _tc_head1(aggc, x2p, W1, b1, W2, b2):
    ncin = aggc.shape[0]
    Fo = W2.shape[0]
    grid = (N // _R,)
    return pl.pallas_call(
        _head1_body,
        grid=grid,
        in_specs=[
            pl.BlockSpec((ncin, _R, C), lambda i: (0, i, 0)),
            pl.BlockSpec((_R, x2p.shape[1]), lambda i: (i, 0)),
            _full(W1.shape), _full(b1.shape), _full(W2.shape), _full(b2.shape),
        ],
        out_specs=[
            pl.BlockSpec((_R, Fo), lambda i: (i, 0)),
            pl.BlockSpec((8, Fo), lambda i: (0, 0)),
        ],
        out_shape=[
            jax.ShapeDtypeStruct((N, Fo), jnp.float32),
            jax.ShapeDtypeStruct((8, Fo), jnp.float32),
        ],
    )(aggc, x2p, W1, b1, W2, b2)


def _tc_head2(u, scale, shift, W, b):
    Fo = W.shape[0]
    grid = (N // _R,)
    return pl.pallas_call(
        _head2_body,
        grid=grid,
        in_specs=[
            pl.BlockSpec((_R, u.shape[1]), lambda i: (i, 0)),
            _full(scale.shape), _full(shift.shape),
            _full(W.shape), _full(b.shape),
        ],
        out_specs=pl.BlockSpec((_R, Fo), lambda i: (i, 0)),
        out_shape=jax.ShapeDtypeStruct((N, Fo), jnp.float32),
    )(u, scale, shift, W, b)


def _sc_agg(hflat, src, dst, w, nc):
    return _make_sc_agg(nc)(hflat, src, dst, w)


def kernel(x, edge_weight, W1, b1, W1s, b1s, W2, b2, W2s, b2s, W3, b3,
           W3s, b3s, Wfc1, bfc1, Wfc2a, bfc2a, gamma, beta, Wfc2b, bfc2b,
           edge_index):
    f32 = jnp.float32
    src = edge_index[1].astype(jnp.int32)
    dstn = edge_index[0].astype(jnp.int32)
    npad = E_PAD - src.shape[0]
    srcp = jnp.concatenate([src, jnp.zeros((npad,), jnp.int32)])
    dstp = jnp.concatenate([dstn, jnp.zeros((npad,), jnp.int32)])
    wp = jnp.concatenate([edge_weight.astype(f32), jnp.zeros((npad,), f32)])

    r2 = lambda v: v.reshape(1, -1)

    hc1, x21 = _tc_layer1(x, W1, r2(b1), W1s, r2(b1s), nc=7)
    agg1 = _sc_agg(hc1.reshape(7 * N, C), srcp, dstp, wp, 7)

    W2p = jnp.pad(W2, ((0, 0), (0, 24)))
    W2sp = jnp.pad(W2s, ((0, 0), (0, 24)))
    hc2, x22 = _tc_layerB(agg1, x21, W2p, r2(b2), W2sp, r2(b2s), nc=4)
    agg2 = _sc_agg(hc2.reshape(4 * N, C), srcp, dstp, wp, 4)

    hc3, x23 = _tc_layerB(agg2, x22, W3, r2(b3), W3s, r2(b3s), nc=4)
    agg3 = _sc_agg(hc3.reshape(4 * N, C), srcp, dstp, wp, 4)

    u, st = _tc_head1(agg3, x23, Wfc1, r2(bfc1), Wfc2a, r2(bfc2a))
    mean = st[0] / N
    var = st[1] / N - mean * mean
    scale = gamma / jnp.sqrt(var + EPS)
    shift = beta - mean * scale
    return _tc_head2(u, r2(scale), r2(shift), Wfc2b, r2(bfc2b))


# speedup vs baseline: 3.2377x; 1.0015x over previous
import functools

import jax
import jax.numpy as jnp
from jax import lax
from jax.experimental import pallas as pl
from jax.experimental.pallas import tpu as pltpu
from jax.experimental.pallas import tpu_sc as plsc

N = 50000
EPS = 1e-5
C = 32
LANES = 16
EB = 128
SUPER = 1024
NTILES = 16
ROWS_PT = 3128
NPAD = NTILES * ROWS_PT
PT = 50176
E_PAD = NTILES * PT
NSB = PT // SUPER


@functools.lru_cache(maxsize=None)
def _make_sc_agg(nc):
    mesh = plsc.VectorSubcoreMesh(core_axis_name="c", subcore_axis_name="s")

    @functools.partial(
        pl.kernel,
        mesh=mesh,
        compiler_params=pltpu.CompilerParams(use_tc_tiling_on_sc=False),
        out_type=jax.ShapeDtypeStruct((nc, NPAD, C), jnp.float32),
        scratch_types=[
            pltpu.VMEM_SHARED((NPAD, C), jnp.float32),
            pltpu.VMEM((SUPER,), jnp.int32),
            pltpu.VMEM((SUPER,), jnp.int32),
            pltpu.VMEM((SUPER,), jnp.float32),
            pltpu.VMEM((EB,), jnp.int32),
            pltpu.VMEM((EB,), jnp.int32),
            pltpu.VMEM((EB,), jnp.int32),
            pltpu.VMEM((EB,), jnp.int32),
            pltpu.VMEM((EB,), jnp.int32),
            pltpu.VMEM((EB,), jnp.int32),
            pltpu.VMEM((EB, C), jnp.float32),
            pltpu.VMEM((EB, C), jnp.float32),
            pltpu.VMEM((EB, C), jnp.float32),
            pltpu.SemaphoreType.DMA,
            pltpu.SemaphoreType.DMA,
            pltpu.SemaphoreType.DMA,
            pltpu.SemaphoreType.DMA,
            pltpu.SemaphoreType.DMA,
            pltpu.SemaphoreType.DMA,
        ],
    )
    def sc_agg(hflat, src, dst, w, out, acc, srcb, dstb, wb,
               gidx0, gidx1, gidx2, sidx0, sidx1, sidx2,
               rows0, rows1, rows2, gs0, gs1, gs2, ss0, ss1, ss2):
        gx = (gidx0, gidx1, gidx2)
        sx = (sidx0, sidx1, sidx2)
        rw = (rows0, rows1, rows2)
        gsm = (gs0, gs1, gs2)
        ssm = (ss0, ss1, ss2)
        cid = lax.axis_index("c")
        sid = lax.axis_index("s")
        ebase = sid * PT
        r0 = sid * ROWS_PT
        nch = (nc - cid + 1) // 2

        def chunk_body(i, carry):
            ch = 2 * i + cid
            zv = jnp.zeros((LANES,), jnp.float32)

            def zrow(e, c2):
                rows0[e, pl.ds(0, LANES)] = zv
                rows0[e, pl.ds(LANES, LANES)] = zv
                return c2

            lax.fori_loop(0, EB, zrow, 0)
            nfull = ROWS_PT // EB
            rem = ROWS_PT - nfull * EB

            def zcp(z, c2):
                pltpu.sync_copy(rows0, acc.at[pl.ds(r0 + z * EB, EB)])
                return c2

            lax.fori_loop(0, nfull, zcp, 0)
            pltpu.sync_copy(rows0.at[pl.ds(0, rem)],
                            acc.at[pl.ds(r0 + nfull * EB, rem)])
            plsc.subcore_barrier()

            chn = jnp.full((LANES,), ch * N, jnp.int32)

            nb = SUPER // EB

            def super_body(sb, c2):
                off = ebase + sb * SUPER
                pltpu.sync_copy(src.at[pl.ds(off, SUPER)], srcb)
                pltpu.sync_copy(dst.at[pl.ds(off, SUPER)], dstb)
                pltpu.sync_copy(w.at[pl.ds(off, SUPER)], wb)

                def build(b, p):
                    for v in range(EB // LANES):
                        gx[p][pl.ds(v * LANES, LANES)] = (
                            srcb[pl.ds(b * EB + v * LANES, LANES)] + chn)
                        sx[p][pl.ds(v * LANES, LANES)] = (
                            dstb[pl.ds(b * EB + v * LANES, LANES)])

                build(0, 0)
                G = [None, None, None]
                S = [None, None, None]
                G[0] = pltpu.async_copy(hflat.at[gx[0]], rw[0], gsm[0])
                for b in range(nb):
                    p = b % 3
                    if b + 1 < nb:
                        q = (b + 1) % 3
                        if S[q] is not None:
                            S[q].wait()
                        build(b + 1, q)
                        G[q] = pltpu.async_copy(hflat.at[gx[q]], rw[q], gsm[q])
                    G[p].wait()
                    rows = rw[p]
                    for g in range(EB // LANES):
                        wv = wb[pl.ds(b * EB + g * LANES, LANES)]
                        for l in range(LANES):
                            e = g * LANES + l
                            ws = wv[l]
                            rows[e, pl.ds(0, LANES)] = (
                                rows[e, pl.ds(0, LANES)] * ws)
                            rows[e, pl.ds(LANES, LANES)] = (
                                rows[e, pl.ds(LANES, LANES)] * ws)
                    S[p] = pltpu.async_copy(rows, acc.at[sx[p]], ssm[p],
                                            add=True)
                for b in range(nb - 3, nb):
                    S[b % 3].wait()
                return c2

            lax.fori_loop(0, NSB, super_body, 0)
            plsc.subcore_barrier()
            pltpu.sync_copy(acc.at[pl.ds(r0, ROWS_PT)],
                            out.at[ch, pl.ds(r0, ROWS_PT)])
            plsc.subcore_barrier()
            return carry

        lax.fori_loop(0, nch, chunk_body, 0)

    return sc_agg


_R = 2000


def _dot(a, b):
    return lax.dot_general(a, b, (((1,), (1,)), ((), ())),
                           preferred_element_type=jnp.float32)


def _l1_body(x_ref, W_ref, b_ref, Ws_ref, bs_ref, hc_ref, x2_ref):
    xb = x_ref[...]
    h = _dot(xb, W_ref[...]) + b_ref[...]
    x2 = _dot(xb, Ws_ref[...]) + bs_ref[...]
    nc = hc_ref.shape[0]
    hp = jnp.pad(h, ((0, 0), (0, nc * C - h.shape[1])))
    x2_ref[...] = jnp.pad(x2, ((0, 0), (0, x2_ref.shape[1] - x2.shape[1])))
    for c in range(nc):
        hc_ref[c] = hp[:, c * C:(c + 1) * C]


def _lB_body(aggc_ref, x2p_ref, W_ref, b_ref, Ws_ref, bs_ref, hc_ref, x2_ref):
    ncin = aggc_ref.shape[0]
    hin = jnp.concatenate(
        [jax.nn.relu(aggc_ref[c] + x2p_ref[:, c * C:(c + 1) * C])
         for c in range(ncin)], axis=1)
    h = _dot(hin, W_ref[...]) + b_ref[...]
    x2 = _dot(hin, Ws_ref[...]) + bs_ref[...]
    nc = hc_ref.shape[0]
    hp = jnp.pad(h, ((0, 0), (0, nc * C - h.shape[1])))
    x2_ref[...] = jnp.pad(x2, ((0, 0), (0, x2_ref.shape[1] - x2.shape[1])))
    for c in range(nc):
        hc_ref[c] = hp[:, c * C:(c + 1) * C]


def _head1_body(aggc_ref, x2p_ref, W1_ref, b1_ref, W2_ref, b2_ref,
                u_ref, st_ref):
    i = pl.program_id(0)
    ncin = aggc_ref.shape[0]
    h3 = jnp.concatenate(
        [jax.nn.relu(aggc_ref[c] + x2p_ref[:, c * C:(c + 1) * C])
         for c in range(ncin)], axis=1)
    t = jax.nn.relu(_dot(h3, W1_ref[...]) + b1_ref[...])
    u = _dot(t, W2_ref[...]) + b2_ref[...]
    u_ref[...] = u
    s0 = jnp.sum(u, axis=0, keepdims=True)
    s1 = jnp.sum(u * u, axis=0, keepdims=True)
    upd = jnp.concatenate([s0, s1, jnp.zeros((6, u.shape[1]), jnp.float32)],
                          axis=0)

    @pl.when(i == 0)
    def _():
        st_ref[...] = upd

    @pl.when(i > 0)
    def _():
        st_ref[...] = st_ref[...] + upd


def _head2_body(u_ref, sc_ref, sh_ref, W_ref, b_ref, o_ref):
    y = jax.nn.relu(u_ref[...] * sc_ref[...] + sh_ref[...])
    o_ref[...] = _dot(y, W_ref[...]) + b_ref[...]


def _full(shape):
    return pl.BlockSpec(shape, lambda i: (0,) * len(shape))


def _tc_layer1(x, W, b, Ws, bs, nc):
    Fop = nc * C
    grid = (N // _R,)
    return pl.pallas_call(
        _l1_body,
        grid=grid,
        in_specs=[
            pl.BlockSpec((_R, x.shape[1]), lambda i: (i, 0)),
            _full(W.shape), _full(b.shape), _full(Ws.shape), _full(bs.shape),
        ],
        out_specs=[
            pl.BlockSpec((nc, _R, C), lambda i: (0, i, 0)),
            pl.BlockSpec((_R, Fop), lambda i: (i, 0)),
        ],
        out_shape=[
            jax.ShapeDtypeStruct((nc, N, C), jnp.float32),
            jax.ShapeDtypeStruct((N, Fop), jnp.float32),
        ],
    )(x, W, b, Ws, bs)


def _tc_layerB(aggc, x2p, W, b, Ws, bs, nc):
    ncin = aggc.shape[0]
    Fop = nc * C
    grid = (N // _R,)
    return pl.pallas_call(
        _lB_body,
        grid=grid,
        in_specs=[
            pl.BlockSpec((ncin, _R, C), lambda i: (0, i, 0)),
            pl.BlockSpec((_R, x2p.shape[1]), lambda i: (i, 0)),
            _full(W.shape), _full(b.shape), _full(Ws.shape), _full(bs.shape),
        ],
        out_specs=[
            pl.BlockSpec((nc, _R, C), lambda i: (0, i, 0)),
            pl.BlockSpec((_R, Fop), lambda i: (i, 0)),
        ],
        out_shape=[
            jax.ShapeDtypeStruct((nc, N, C), jnp.float32),
            jax.ShapeDtypeStruct((N, Fop), jnp.float32),
        ],
    )(aggc, x2p, W, b, Ws, bs)


def _tc_head1(aggc, x2p, W1, b1, W2, b2):
    ncin = aggc.shape[0]
    Fo = W2.shape[0]
    grid = (N // _R,)
    return pl.pallas_call(
        _head1_body,
        grid=grid,
        in_specs=[
            pl.BlockSpec((ncin, _R, C), lambda i: (0, i, 0)),
            pl.BlockSpec((_R, x2p.shape[1]), lambda i: (i, 0)),
            _full(W1.shape), _full(b1.shape), _full(W2.shape), _full(b2.shape),
        ],
        out_specs=[
            pl.BlockSpec((_R, Fo), lambda i: (i, 0)),
            pl.BlockSpec((8, Fo), lambda i: (0, 0)),
        ],
        out_shape=[
            jax.ShapeDtypeStruct((N, Fo), jnp.float32),
            jax.ShapeDtypeStruct((8, Fo), jnp.float32),
        ],
    )(aggc, x2p, W1, b1, W2, b2)


def _tc_head2(u, scale, shift, W, b):
    Fo = W.shape[0]
    grid = (N // _R,)
    return pl.pallas_call(
        _head2_body,
        grid=grid,
        in_specs=[
            pl.BlockSpec((_R, u.shape[1]), lambda i: (i, 0)),
            _full(scale.shape), _full(shift.shape),
            _full(W.shape), _full(b.shape),
        ],
        out_specs=pl.BlockSpec((_R, Fo), lambda i: (i, 0)),
        out_shape=jax.ShapeDtypeStruct((N, Fo), jnp.float32),
    )(u, scale, shift, W, b)


def _sc_agg(hflat, src, dst, w, nc):
    return _make_sc_agg(nc)(hflat, src, dst, w)


def kernel(x, edge_weight, W1, b1, W1s, b1s, W2, b2, W2s, b2s, W3, b3,
           W3s, b3s, Wfc1, bfc1, Wfc2a, bfc2a, gamma, beta, Wfc2b, bfc2b,
           edge_index):
    f32 = jnp.float32
    src = edge_index[1].astype(jnp.int32)
    dstn = edge_index[0].astype(jnp.int32)
    npad = E_PAD - src.shape[0]
    srcp = jnp.concatenate([src, jnp.zeros((npad,), jnp.int32)])
    dstp = jnp.concatenate([dstn, jnp.zeros((npad,), jnp.int32)])
    wp = jnp.concatenate([edge_weight.astype(f32), jnp.zeros((npad,), f32)])

    r2 = lambda v: v.reshape(1, -1)

    hc1, x21 = _tc_layer1(x, W1, r2(b1), W1s, r2(b1s), nc=7)
    agg1 = _sc_agg(hc1.reshape(7 * N, C), srcp, dstp, wp, 7)

    W2p = jnp.pad(W2, ((0, 0), (0, 24)))
    W2sp = jnp.pad(W2s, ((0, 0), (0, 24)))
    hc2, x22 = _tc_layerB(agg1, x21, W2p, r2(b2), W2sp, r2(b2s), nc=4)
    agg2 = _sc_agg(hc2.reshape(4 * N, C), srcp, dstp, wp, 4)

    hc3, x23 = _tc_layerB(agg2, x22, W3, r2(b3), W3s, r2(b3s), nc=4)
    agg3 = _sc_agg(hc3.reshape(4 * N, C), srcp, dstp, wp, 4)

    u, st = _tc_head1(agg3, x23, Wfc1, r2(bfc1), Wfc2a, r2(bfc2a))
    mean = st[0] / N
    var = st[1] / N - mean * mean
    scale = gamma / jnp.sqrt(var + EPS)
    shift = beta - mean * scale
    return _tc_head2(u, r2(scale), r2(shift), Wfc2b, r2(bfc2b))


# speedup vs baseline: 3.2413x; 1.0011x over previous
import functools

import jax
import jax.numpy as jnp
from jax import lax
from jax.experimental import pallas as pl
from jax.experimental.pallas import tpu as pltpu
from jax.experimental.pallas import tpu_sc as plsc

N = 50000
EPS = 1e-5
C = 32
LANES = 16
EB = 128
SUPER = 1024
NTILES = 16
ROWS_PT = 3128
NPAD = NTILES * ROWS_PT
PT = 50176
E_PAD = NTILES * PT
NSB = PT // SUPER


@functools.lru_cache(maxsize=None)
def _make_sc_agg(nc):
    mesh = plsc.VectorSubcoreMesh(core_axis_name="c", subcore_axis_name="s")

    @functools.partial(
        pl.kernel,
        mesh=mesh,
        compiler_params=pltpu.CompilerParams(use_tc_tiling_on_sc=False),
        out_type=jax.ShapeDtypeStruct((nc, NPAD, C), jnp.float32),
        scratch_types=[
            pltpu.VMEM_SHARED((NPAD, C), jnp.float32),
            pltpu.VMEM((SUPER,), jnp.int32),
            pltpu.VMEM((SUPER,), jnp.int32),
            pltpu.VMEM((SUPER,), jnp.float32),
            pltpu.VMEM((EB,), jnp.int32),
            pltpu.VMEM((EB,), jnp.int32),
            pltpu.VMEM((EB,), jnp.int32),
            pltpu.VMEM((EB,), jnp.int32),
            pltpu.VMEM((EB,), jnp.int32),
            pltpu.VMEM((EB,), jnp.int32),
            pltpu.VMEM((EB, C), jnp.float32),
            pltpu.VMEM((EB, C), jnp.float32),
            pltpu.VMEM((EB, C), jnp.float32),
            pltpu.VMEM((EB, C), jnp.float32),
            pltpu.VMEM((EB, C), jnp.float32),
            pltpu.VMEM((EB, C), jnp.float32),
            pltpu.SemaphoreType.DMA,
            pltpu.SemaphoreType.DMA,
            pltpu.SemaphoreType.DMA,
            pltpu.SemaphoreType.DMA,
            pltpu.SemaphoreType.DMA,
            pltpu.SemaphoreType.DMA,
        ],
    )
    def sc_agg(hflat, src, dst, w, out, acc, srcb, dstb, wb,
               gidx0, gidx1, gidx2, sidx0, sidx1, sidx2,
               rows0, rows1, rows2, sc0, sc1, sc2, gs0, gs1, gs2,
               ss0, ss1, ss2):
        gx = (gidx0, gidx1, gidx2)
        sx = (sidx0, sidx1, sidx2)
        rw = (rows0, rows1, rows2)
        sos = (sc0, sc1, sc2)
        gsm = (gs0, gs1, gs2)
        ssm = (ss0, ss1, ss2)
        cid = lax.axis_index("c")
        sid = lax.axis_index("s")
        ebase = sid * PT
        r0 = sid * ROWS_PT
        nch = (nc - cid + 1) // 2

        def chunk_body(i, carry):
            ch = 2 * i + cid
            zv = jnp.zeros((LANES,), jnp.float32)

            def zrow(e, c2):
                rows0[e, pl.ds(0, LANES)] = zv
                rows0[e, pl.ds(LANES, LANES)] = zv
                return c2

            lax.fori_loop(0, EB, zrow, 0)
            nfull = ROWS_PT // EB
            rem = ROWS_PT - nfull * EB

            def zcp(z, c2):
                pltpu.sync_copy(rows0, acc.at[pl.ds(r0 + z * EB, EB)])
                return c2

            lax.fori_loop(0, nfull, zcp, 0)
            pltpu.sync_copy(rows0.at[pl.ds(0, rem)],
                            acc.at[pl.ds(r0 + nfull * EB, rem)])
            plsc.subcore_barrier()

            chn = jnp.full((LANES,), ch * N, jnp.int32)

            nb = SUPER // EB

            def super_body(sb, c2):
                off = ebase + sb * SUPER
                pltpu.sync_copy(src.at[pl.ds(off, SUPER)], srcb)
                pltpu.sync_copy(dst.at[pl.ds(off, SUPER)], dstb)
                pltpu.sync_copy(w.at[pl.ds(off, SUPER)], wb)

                def build(b, p):
                    for v in range(EB // LANES):
                        gx[p][pl.ds(v * LANES, LANES)] = (
                            srcb[pl.ds(b * EB + v * LANES, LANES)] + chn)
                        sx[p][pl.ds(v * LANES, LANES)] = (
                            dstb[pl.ds(b * EB + v * LANES, LANES)])

                build(0, 0)
                G = [None, None, None]
                S = [None, None, None]
                G[0] = pltpu.async_copy(hflat.at[gx[0]], rw[0], gsm[0])
                for b in range(nb):
                    p = b % 3
                    if b + 1 < nb:
                        q = (b + 1) % 3
                        if S[q] is not None:
                            S[q].wait()
                        build(b + 1, q)
                        G[q] = pltpu.async_copy(hflat.at[gx[q]], rw[q], gsm[q])
                    G[p].wait()
                    rows = rw[p]
                    so = sos[p]
                    wvs = [wb[pl.ds(b * EB + g * LANES, LANES)]
                           for g in range(EB // LANES)]
                    wss = [None] * EB
                    vals = [None] * (2 * EB)
                    D = 8
                    for i in range(2 * EB + D):
                        if i < 2 * EB:
                            e, h = divmod(i, 2)
                            vals[i] = rows[e, pl.ds(h * LANES, LANES)]
                        if i >= D:
                            e, h = divmod(i - D, 2)
                            if wss[e] is None:
                                wss[e] = wvs[e // LANES][e % LANES]
                            so[e, pl.ds(h * LANES, LANES)] = (
                                vals[i - D] * wss[e])
                    S[p] = pltpu.async_copy(so, acc.at[sx[p]], ssm[p],
                                            add=True)
                for b in range(nb - 3, nb):
                    S[b % 3].wait()
                return c2

            lax.fori_loop(0, NSB, super_body, 0)
            plsc.subcore_barrier()
            pltpu.sync_copy(acc.at[pl.ds(r0, ROWS_PT)],
                            out.at[ch, pl.ds(r0, ROWS_PT)])
            plsc.subcore_barrier()
            return carry

        lax.fori_loop(0, nch, chunk_body, 0)

    return sc_agg


_R = 2000


def _dot(a, b):
    return lax.dot_general(a, b, (((1,), (1,)), ((), ())),
                           preferred_element_type=jnp.float32)


def _l1_body(x_ref, W_ref, b_ref, Ws_ref, bs_ref, hc_ref, x2_ref):
    xb = x_ref[...]
    h = _dot(xb, W_ref[...]) + b_ref[...]
    x2 = _dot(xb, Ws_ref[...]) + bs_ref[...]
    nc = hc_ref.shape[0]
    hp = jnp.pad(h, ((0, 0), (0, nc * C - h.shape[1])))
    x2_ref[...] = jnp.pad(x2, ((0, 0), (0, x2_ref.shape[1] - x2.shape[1])))
    for c in range(nc):
        hc_ref[c] = hp[:, c * C:(c + 1) * C]


def _lB_body(aggc_ref, x2p_ref, W_ref, b_ref, Ws_ref, bs_ref, hc_ref, x2_ref):
    ncin = aggc_ref.shape[0]
    hin = jnp.concatenate(
        [jax.nn.relu(aggc_ref[c] + x2p_ref[:, c * C:(c + 1) * C])
         for c in range(ncin)], axis=1)
    h = _dot(hin, W_ref[...]) + b_ref[...]
    x2 = _dot(hin, Ws_ref[...]) + bs_ref[...]
    nc = hc_ref.shape[0]
    hp = jnp.pad(h, ((0, 0), (0, nc * C - h.shape[1])))
    x2_ref[...] = jnp.pad(x2, ((0, 0), (0, x2_ref.shape[1] - x2.shape[1])))
    for c in range(nc):
        hc_ref[c] = hp[:, c * C:(c + 1) * C]


def _head1_body(aggc_ref, x2p_ref, W1_ref, b1_ref, W2_ref, b2_ref,
                u_ref, st_ref):
    i = pl.program_id(0)
    ncin = aggc_ref.shape[0]
    h3 = jnp.concatenate(
        [jax.nn.relu(aggc_ref[c] + x2p_ref[:, c * C:(c + 1) * C])
         for c in range(ncin)], axis=1)
    t = jax.nn.relu(_dot(h3, W1_ref[...]) + b1_ref[...])
    u = _dot(t, W2_ref[...]) + b2_ref[...]
    u_ref[...] = u
    s0 = jnp.sum(u, axis=0, keepdims=True)
    s1 = jnp.sum(u * u, axis=0, keepdims=True)
    upd = jnp.concatenate([s0, s1, jnp.zeros((6, u.shape[1]), jnp.float32)],
                          axis=0)

    @pl.when(i == 0)
    def _():
        st_ref[...] = upd

    @pl.when(i > 0)
    def _():
        st_ref[...] = st_ref[...] + upd


def _head2_body(u_ref, sc_ref, sh_ref, W_ref, b_ref, o_ref):
    y = jax.nn.relu(u_ref[...] * sc_ref[...] + sh_ref[...])
    o_ref[...] = _dot(y, W_ref[...]) + b_ref[...]


def _full(shape):
    return pl.BlockSpec(shape, lambda i: (0,) * len(shape))


def _tc_layer1(x, W, b, Ws, bs, nc):
    Fop = nc * C
    grid = (N // _R,)
    return pl.pallas_call(
        _l1_body,
        grid=grid,
        in_specs=[
            pl.BlockSpec((_R, x.shape[1]), lambda i: (i, 0)),
            _full(W.shape), _full(b.shape), _full(Ws.shape), _full(bs.shape),
        ],
        out_specs=[
            pl.BlockSpec((nc, _R, C), lambda i: (0, i, 0)),
            pl.BlockSpec((_R, Fop), lambda i: (i, 0)),
        ],
        out_shape=[
            jax.ShapeDtypeStruct((nc, N, C), jnp.float32),
            jax.ShapeDtypeStruct((N, Fop), jnp.float32),
        ],
    )(x, W, b, Ws, bs)


def _tc_layerB(aggc, x2p, W, b, Ws, bs, nc):
    ncin = aggc.shape[0]
    Fop = nc * C
    grid = (N // _R,)
    return pl.pallas_call(
        _lB_body,
        grid=grid,
        in_specs=[
            pl.BlockSpec((ncin, _R, C), lambda i: (0, i, 0)),
            pl.BlockSpec((_R, x2p.shape[1]), lambda i: (i, 0)),
            _full(W.shape), _full(b.shape), _full(Ws.shape), _full(bs.shape),
        ],
        out_specs=[
            pl.BlockSpec((nc, _R, C), lambda i: (0, i, 0)),
            pl.BlockSpec((_R, Fop), lambda i: (i, 0)),
        ],
        out_shape=[
            jax.ShapeDtypeStruct((nc, N, C), jnp.float32),
            jax.ShapeDtypeStruct((N, Fop), jnp.float32),
        ],
    )(aggc, x2p, W, b, Ws, bs)


def _tc_head1(aggc, x2p, W1, b1, W2, b2):
    ncin = aggc.shape[0]
    Fo = W2.shape[0]
    grid = (N // _R,)
    return pl.pallas_call(
        _head1_body,
        grid=grid,
        in_specs=[
            pl.BlockSpec((ncin, _R, C), lambda i: (0, i, 0)),
            pl.BlockSpec((_R, x2p.shape[1]), lambda i: (i, 0)),
            _full(W1.shape), _full(b1.shape), _full(W2.shape), _full(b2.shape),
        ],
        out_specs=[
            pl.BlockSpec((_R, Fo), lambda i: (i, 0)),
            pl.BlockSpec((8, Fo), lambda i: (0, 0)),
        ],
        out_shape=[
            jax.ShapeDtypeStruct((N, Fo), jnp.float32),
            jax.ShapeDtypeStruct((8, Fo), jnp.float32),
        ],
    )(aggc, x2p, W1, b1, W2, b2)


def _tc_head2(u, scale, shift, W, b):
    Fo = W.shape[0]
    grid = (N // _R,)
    return pl.pallas_call(
        _head2_body,
        grid=grid,
        in_specs=[
            pl.BlockSpec((_R, u.shape[1]), lambda i: (i, 0)),
            _full(scale.shape), _full(shift.shape),
            _full(W.shape), _full(b.shape),
        ],
        out_specs=pl.BlockSpec((_R, Fo), lambda i: (i, 0)),
        out_shape=jax.ShapeDtypeStruct((N, Fo), jnp.float32),
    )(u, scale, shift, W, b)


def _sc_agg(hflat, src, dst, w, nc):
    return _make_sc_agg(nc)(hflat, src, dst, w)


def kernel(x, edge_weight, W1, b1, W1s, b1s, W2, b2, W2s, b2s, W3, b3,
           W3s, b3s, Wfc1, bfc1, Wfc2a, bfc2a, gamma, beta, Wfc2b, bfc2b,
           edge_index):
    f32 = jnp.float32
    src = edge_index[1].astype(jnp.int32)
    dstn = edge_index[0].astype(jnp.int32)
    npad = E_PAD - src.shape[0]
    srcp = jnp.concatenate([src, jnp.zeros((npad,), jnp.int32)])
    dstp = jnp.concatenate([dstn, jnp.zeros((npad,), jnp.int32)])
    wp = jnp.concatenate([edge_weight.astype(f32), jnp.zeros((npad,), f32)])

    r2 = lambda v: v.reshape(1, -1)

    hc1, x21 = _tc_layer1(x, W1, r2(b1), W1s, r2(b1s), nc=7)
    agg1 = _sc_agg(hc1.reshape(7 * N, C), srcp, dstp, wp, 7)

    W2p = jnp.pad(W2, ((0, 0), (0, 24)))
    W2sp = jnp.pad(W2s, ((0, 0), (0, 24)))
    hc2, x22 = _tc_layerB(agg1, x21, W2p, r2(b2), W2sp, r2(b2s), nc=4)
    agg2 = _sc_agg(hc2.reshape(4 * N, C), srcp, dstp, wp, 4)

    hc3, x23 = _tc_layerB(agg2, x22, W3, r2(b3), W3s, r2(b3s), nc=4)
    agg3 = _sc_agg(hc3.reshape(4 * N, C), srcp, dstp, wp, 4)

    u, st = _tc_head1(agg3, x23, Wfc1, r2(bfc1), Wfc2a, r2(bfc2a))
    mean = st[0] / N
    var = st[1] / N - mean * mean
    scale = gamma / jnp.sqrt(var + EPS)
    shift = beta - mean * scale
    return _tc_head2(u, r2(scale), r2(shift), Wfc2b, r2(bfc2b))


# speedup vs baseline: 3.2927x; 1.0159x over previous
import functools

import jax
import jax.numpy as jnp
from jax import lax
from jax.experimental import pallas as pl
from jax.experimental.pallas import tpu as pltpu
from jax.experimental.pallas import tpu_sc as plsc

N = 50000
EPS = 1e-5
C = 32
LANES = 16
EB = 128
SUPER = 1024
NTILES = 16
ROWS_PT = 3128
NPAD = NTILES * ROWS_PT
PT = 50176
E_PAD = NTILES * PT
NSB = PT // SUPER


@functools.lru_cache(maxsize=None)
def _make_sc_agg(nc):
    mesh = plsc.VectorSubcoreMesh(core_axis_name="c", subcore_axis_name="s")

    @functools.partial(
        pl.kernel,
        mesh=mesh,
        compiler_params=pltpu.CompilerParams(use_tc_tiling_on_sc=False),
        out_type=jax.ShapeDtypeStruct((nc, NPAD, C), jnp.float32),
        scratch_types=[
            pltpu.VMEM_SHARED((NPAD, C), jnp.float32),
            pltpu.VMEM((SUPER,), jnp.int32),
            pltpu.VMEM((SUPER,), jnp.int32),
            pltpu.VMEM((SUPER,), jnp.float32),
            pltpu.VMEM((EB,), jnp.int32),
            pltpu.VMEM((EB,), jnp.int32),
            pltpu.VMEM((EB,), jnp.int32),
            pltpu.VMEM((EB,), jnp.int32),
            pltpu.VMEM((EB,), jnp.int32),
            pltpu.VMEM((EB,), jnp.int32),
            pltpu.VMEM((EB, C), jnp.float32),
            pltpu.VMEM((EB, C), jnp.float32),
            pltpu.VMEM((EB, C), jnp.float32),
            pltpu.VMEM((EB, C), jnp.float32),
            pltpu.VMEM((EB, C), jnp.float32),
            pltpu.VMEM((EB, C), jnp.float32),
            pltpu.SemaphoreType.DMA,
            pltpu.SemaphoreType.DMA,
            pltpu.SemaphoreType.DMA,
            pltpu.SemaphoreType.DMA,
            pltpu.SemaphoreType.DMA,
            pltpu.SemaphoreType.DMA,
        ],
    )
    def sc_agg(hflat, src, dst, w, out, acc, srcb, dstb, wb,
               gidx0, gidx1, gidx2, sidx0, sidx1, sidx2,
               rows0, rows1, rows2, sc0, sc1, sc2, gs0, gs1, gs2,
               ss0, ss1, ss2):
        gx = (gidx0, gidx1, gidx2)
        sx = (sidx0, sidx1, sidx2)
        rw = (rows0, rows1, rows2)
        sos = (sc0, sc1, sc2)
        gsm = (gs0, gs1, gs2)
        ssm = (ss0, ss1, ss2)
        cid = lax.axis_index("c")
        sid = lax.axis_index("s")
        ebase = sid * PT
        r0 = sid * ROWS_PT
        nch = (nc - cid + 1) // 2

        def chunk_body(i, carry):
            ch = 2 * i + cid
            zv = jnp.zeros((LANES,), jnp.float32)

            def zrow(e, c2):
                rows0[e, pl.ds(0, LANES)] = zv
                rows0[e, pl.ds(LANES, LANES)] = zv
                return c2

            lax.fori_loop(0, EB, zrow, 0)
            nfull = ROWS_PT // EB
            rem = ROWS_PT - nfull * EB

            def zcp(z, c2):
                pltpu.sync_copy(rows0, acc.at[pl.ds(r0 + z * EB, EB)])
                return c2

            lax.fori_loop(0, nfull, zcp, 0)
            pltpu.sync_copy(rows0.at[pl.ds(0, rem)],
                            acc.at[pl.ds(r0 + nfull * EB, rem)])
            plsc.subcore_barrier()

            chn = jnp.full((LANES,), ch * N, jnp.int32)

            nb = SUPER // EB

            def super_body(sb, c2):
                off = ebase + sb * SUPER
                pltpu.sync_copy(src.at[pl.ds(off, SUPER)], srcb)
                pltpu.sync_copy(dst.at[pl.ds(off, SUPER)], dstb)
                pltpu.sync_copy(w.at[pl.ds(off, SUPER)], wb)

                def build(b, p):
                    for v in range(EB // LANES):
                        gx[p][pl.ds(v * LANES, LANES)] = (
                            srcb[pl.ds(b * EB + v * LANES, LANES)] + chn)
                        sx[p][pl.ds(v * LANES, LANES)] = (
                            dstb[pl.ds(b * EB + v * LANES, LANES)])

                build(0, 0)
                G = [None, None, None]
                S = [None, None, None]
                G[0] = pltpu.async_copy(hflat.at[gx[0]], rw[0], gsm[0])
                for b in range(nb):
                    p = b % 3
                    if b + 1 < nb:
                        q = (b + 1) % 3
                        if S[q] is not None:
                            S[q].wait()
                        build(b + 1, q)
                        G[q] = pltpu.async_copy(hflat.at[gx[q]], rw[q], gsm[q])
                    G[p].wait()
                    rows = rw[p]
                    so = sos[p]
                    wvs = [wb[pl.ds(b * EB + g * LANES, LANES)]
                           for g in range(EB // LANES)]
                    wss = [None] * EB
                    vals = [None] * (2 * EB)
                    D = 8
                    for i in range(2 * EB + D):
                        if i < 2 * EB:
                            e, h = divmod(i, 2)
                            vals[i] = rows[e, pl.ds(h * LANES, LANES)]
                        if i >= D:
                            e, h = divmod(i - D, 2)
                            if wss[e] is None:
                                wss[e] = wvs[e // LANES][e % LANES]
                            so[e, pl.ds(h * LANES, LANES)] = (
                                vals[i - D] * wss[e])
                    S[p] = pltpu.async_copy(so, acc.at[sx[p]], ssm[p],
                                            add=True)
                for b in range(nb - 3, nb):
                    S[b % 3].wait()
                return c2

            lax.fori_loop(0, NSB, super_body, 0)
            plsc.subcore_barrier()
            pltpu.sync_copy(acc.at[pl.ds(r0, ROWS_PT)],
                            out.at[ch, pl.ds(r0, ROWS_PT)])
            plsc.subcore_barrier()
            return carry

        lax.fori_loop(0, nch, chunk_body, 0)

    return sc_agg


_R = 2000


def _dot(a, b):
    return lax.dot_general(a, b, (((1,), (1,)), ((), ())),
                           preferred_element_type=jnp.float32)


def _l1_body(x_ref, W_ref, b_ref, Ws_ref, bs_ref, hc_ref, x2_ref):
    xb = x_ref[...]
    h = _dot(xb, W_ref[...]) + b_ref[...]
    x2 = _dot(xb, Ws_ref[...]) + bs_ref[...]
    nc = hc_ref.shape[0]
    hp = jnp.pad(h, ((0, 0), (0, nc * C - h.shape[1])))
    x2_ref[...] = jnp.pad(x2, ((0, 0), (0, x2_ref.shape[1] - x2.shape[1])))
    for c in range(nc):
        hc_ref[c] = hp[:, c * C:(c + 1) * C]


def _lB_body(aggc_ref, x2p_ref, W_ref, b_ref, Ws_ref, bs_ref, hc_ref, x2_ref):
    ncin = aggc_ref.shape[0]
    hin = jnp.concatenate(
        [jax.nn.relu(aggc_ref[c] + x2p_ref[:, c * C:(c + 1) * C])
         for c in range(ncin)], axis=1)
    h = _dot(hin, W_ref[...]) + b_ref[...]
    x2 = _dot(hin, Ws_ref[...]) + bs_ref[...]
    nc = hc_ref.shape[0]
    hp = jnp.pad(h, ((0, 0), (0, nc * C - h.shape[1])))
    x2_ref[...] = jnp.pad(x2, ((0, 0), (0, x2_ref.shape[1] - x2.shape[1])))
    for c in range(nc):
        hc_ref[c] = hp[:, c * C:(c + 1) * C]


def _head1_body(aggc_ref, x2p_ref, W1_ref, b1_ref, W2_ref, b2_ref,
                u_ref, st_ref):
    i = pl.program_id(0)
    ncin = aggc_ref.shape[0]
    h3 = jnp.concatenate(
        [jax.nn.relu(aggc_ref[c] + x2p_ref[:, c * C:(c + 1) * C])
         for c in range(ncin)], axis=1)
    t = jax.nn.relu(_dot(h3, W1_ref[...]) + b1_ref[...])
    u = _dot(t, W2_ref[...]) + b2_ref[...]
    u_ref[...] = u
    s0 = jnp.sum(u, axis=0, keepdims=True)
    s1 = jnp.sum(u * u, axis=0, keepdims=True)
    upd = jnp.concatenate([s0, s1, jnp.zeros((6, u.shape[1]), jnp.float32)],
                          axis=0)

    @pl.when(i == 0)
    def _():
        st_ref[...] = upd

    @pl.when(i > 0)
    def _():
        st_ref[...] = st_ref[...] + upd


def _head2_body(u_ref, sc_ref, sh_ref, W_ref, b_ref, o_ref):
    y = jax.nn.relu(u_ref[...] * sc_ref[...] + sh_ref[...])
    o_ref[...] = _dot(y, W_ref[...]) + b_ref[...]


def _full(shape):
    return pl.BlockSpec(shape, lambda i: (0,) * len(shape))


def _tc_layer1(x, W, b, Ws, bs, nc):
    Fop = nc * C
    grid = (N // _R,)
    return pl.pallas_call(
        _l1_body,
        grid=grid,
        in_specs=[
            pl.BlockSpec((_R, x.shape[1]), lambda i: (i, 0)),
            _full(W.shape), _full(b.shape), _full(Ws.shape), _full(bs.shape),
        ],
        out_specs=[
            pl.BlockSpec((nc, _R, C), lambda i: (0, i, 0)),
            pl.BlockSpec((_R, Fop), lambda i: (i, 0)),
        ],
        out_shape=[
            jax.ShapeDtypeStruct((nc, N, C), jnp.float32),
            jax.ShapeDtypeStruct((N, Fop), jnp.float32),
        ],
    )(x, W, b, Ws, bs)


def _tc_layerB(aggc, x2p, W, b, Ws, bs, nc):
    ncin = aggc.shape[0]
    Fop = nc * C
    grid = (N // _R,)
    return pl.pallas_call(
        _lB_body,
        grid=grid,
        in_specs=[
            pl.BlockSpec((ncin, _R, C), lambda i: (0, i, 0)),
            pl.BlockSpec((_R, x2p.shape[1]), lambda i: (i, 0)),
            _full(W.shape), _full(b.shape), _full(Ws.shape), _full(bs.shape),
        ],
        out_specs=[
            pl.BlockSpec((nc, _R, C), lambda i: (0, i, 0)),
            pl.BlockSpec((_R, Fop), lambda i: (i, 0)),
        ],
        out_shape=[
            jax.ShapeDtypeStruct((nc, N, C), jnp.float32),
            jax.ShapeDtypeStruct((N, Fop), jnp.float32),
        ],
    )(aggc, x2p, W, b, Ws, bs)


def _tc_head1(aggc, x2p, W1, b1, W2, b2):
    ncin = aggc.shape[0]
    Fo = W2.shape[0]
    grid = (N // _R,)
    return pl.pallas_call(
        _head1_body,
        grid=grid,
        in_specs=[
            pl.BlockSpec((ncin, _R, C), lambda i: (0, i, 0)),
            pl.BlockSpec((_R, x2p.shape[1]), lambda i: (i, 0)),
            _full(W1.shape), _full(b1.shape), _full(W2.shape), _full(b2.shape),
        ],
        out_specs=[
            pl.BlockSpec((_R, Fo), lambda i: (i, 0)),
            pl.BlockSpec((8, Fo), lambda i: (0, 0)),
        ],
        out_shape=[
            jax.ShapeDtypeStruct((N, Fo), jnp.float32),
            jax.ShapeDtypeStruct((8, Fo), jnp.float32),
        ],
    )(aggc, x2p, W1, b1, W2, b2)


def _tc_head2(u, scale, shift, W, b):
    Fo = W.shape[0]
    grid = (N // _R,)
    return pl.pallas_call(
        _head2_body,
        grid=grid,
        in_specs=[
            pl.BlockSpec((_R, u.shape[1]), lambda i: (i, 0)),
            _full(scale.shape), _full(shift.shape),
            _full(W.shape), _full(b.shape),
        ],
        out_specs=pl.BlockSpec((_R, Fo), lambda i: (i, 0)),
        out_shape=jax.ShapeDtypeStruct((N, Fo), jnp.float32),
    )(u, scale, shift, W, b)


def _sc_agg(hflat, src, dst, w, nc):
    return _make_sc_agg(nc)(hflat, src, dst, w)


def kernel(x, edge_weight, W1, b1, W1s, b1s, W2, b2, W2s, b2s, W3, b3,
           W3s, b3s, Wfc1, bfc1, Wfc2a, bfc2a, gamma, beta, Wfc2b, bfc2b,
           edge_index):
    f32 = jnp.float32
    src = edge_index[1].astype(jnp.int32)
    dstn = edge_index[0].astype(jnp.int32)
    npad = E_PAD - src.shape[0]
    spread = (jnp.arange(npad, dtype=jnp.int32) * 16) % N
    srcp = jnp.concatenate([src, spread])
    dstp = jnp.concatenate([dstn, spread])
    wp = jnp.concatenate([edge_weight.astype(f32), jnp.zeros((npad,), f32)])

    r2 = lambda v: v.reshape(1, -1)

    hc1, x21 = _tc_layer1(x, W1, r2(b1), W1s, r2(b1s), nc=7)
    agg1 = _sc_agg(hc1.reshape(7 * N, C), srcp, dstp, wp, 7)

    W2p = jnp.pad(W2, ((0, 0), (0, 24)))
    W2sp = jnp.pad(W2s, ((0, 0), (0, 24)))
    hc2, x22 = _tc_layerB(agg1, x21, W2p, r2(b2), W2sp, r2(b2s), nc=4)
    agg2 = _sc_agg(hc2.reshape(4 * N, C), srcp, dstp, wp, 4)

    hc3, x23 = _tc_layerB(agg2, x22, W3, r2(b3), W3s, r2(b3s), nc=4)
    agg3 = _sc_agg(hc3.reshape(4 * N, C), srcp, dstp, wp, 4)

    u, st = _tc_head1(agg3, x23, Wfc1, r2(bfc1), Wfc2a, r2(bfc2a))
    mean = st[0] / N
    var = st[1] / N - mean * mean
    scale = gamma / jnp.sqrt(var + EPS)
    shift = beta - mean * scale
    return _tc_head2(u, r2(scale), r2(shift), Wfc2b, r2(bfc2b))
